# baseline (speedup 1.0000x reference)
import functools

import jax
import jax.numpy as jnp
from jax import lax
from jax.experimental import pallas as pl
from jax.experimental.pallas import tpu as pltpu
from jax.experimental.pallas import tpu_sc as plsc

_NEG = -1e30

_T = 2048
_D = 1024
_E = 4
_H = 1368
_B = 256
_G = _T * 2 // _B + (_E - 1)
_P = _G * _B

_NW = 32
_RPW = _P // _NW
_TPW = _T // _NW


def _silu(v):
    return v * (1.0 / (1.0 + jnp.exp(-v)))


def _router_body(x_ref, wg_ref, tw_ref, mask_ref):
    x = x_ref[...]
    logits = jnp.dot(x, wg_ref[...].T, preferred_element_type=jnp.float32)
    m = jnp.max(logits, axis=-1, keepdims=True)
    p = jnp.exp(logits - m)
    p = p / jnp.sum(p, axis=-1, keepdims=True)
    lane = jax.lax.broadcasted_iota(jnp.int32, p.shape, 1)
    p1 = jnp.max(p, axis=-1, keepdims=True)
    i1 = jnp.argmax(p, axis=-1)
    oh1 = (lane == i1[:, None]).astype(jnp.float32)
    pm = jnp.where(oh1 > 0, _NEG, p)
    p2 = jnp.max(pm, axis=-1, keepdims=True)
    i2 = jnp.argmax(pm, axis=-1)
    oh2 = (lane == i2[:, None]).astype(jnp.float32)
    denom = p1 + p2 + 1e-8
    tw_ref[...] = (oh1 * p1 + oh2 * p2) / denom
    mask_ref[...] = oh1 + oh2


def _router(x, Wg):
    return pl.pallas_call(
        _router_body,
        out_shape=(
            jax.ShapeDtypeStruct((_T, _E), jnp.float32),
            jax.ShapeDtypeStruct((_T, _E), jnp.float32),
        ),
    )(x, Wg)


def _sc_gather_body(tok_hbm, x_hbm, xs_hbm, idx_v, rows_v, sem):
    wid = lax.axis_index("s") * 2 + lax.axis_index("c")
    base = wid * _RPW
    for off, n in ((0, 80), (80, 72)):
        pltpu.sync_copy(tok_hbm.at[pl.ds(base + off, n)], idx_v.at[pl.ds(0, n)])
        pltpu.async_copy(
            x_hbm.at[idx_v.at[pl.ds(0, n)]], rows_v.at[pl.ds(0, n)], sem
        ).wait()
        pltpu.sync_copy(rows_v.at[pl.ds(0, n)], xs_hbm.at[pl.ds(base + off, n)])


_sc_gather = functools.partial(
    pl.kernel,
    mesh=plsc.VectorSubcoreMesh(core_axis_name="c", subcore_axis_name="s"),
    out_type=jax.ShapeDtypeStruct((_P, _D), jnp.float32),
    scratch_types=[
        pltpu.VMEM((80,), jnp.int32),
        pltpu.VMEM((80, _D), jnp.float32),
        pltpu.SemaphoreType.DMA,
    ],
)(_sc_gather_body)


def _grouped_body(e_ref, xs_ref, wgate_ref, w1_ref, w2t_ref, ys_ref):
    del e_ref
    xb = xs_ref[...]
    g = _silu(jnp.dot(xb, wgate_ref[0].T, preferred_element_type=jnp.float32))
    u = jnp.dot(xb, w1_ref[0].T, preferred_element_type=jnp.float32)
    ys_ref[...] = jnp.dot(g * u, w2t_ref[0], preferred_element_type=jnp.float32)


def _grouped(xs, W_gate, W1, W2t, e_of_b):
    grid_spec = pltpu.PrefetchScalarGridSpec(
        num_scalar_prefetch=1,
        grid=(_G,),
        in_specs=[
            pl.BlockSpec((_B, _D), lambda b, e: (b, 0)),
            pl.BlockSpec((1, _H, _D), lambda b, e: (e[b], 0, 0)),
            pl.BlockSpec((1, _H, _D), lambda b, e: (e[b], 0, 0)),
            pl.BlockSpec((1, _H, _D), lambda b, e: (e[b], 0, 0)),
        ],
        out_specs=pl.BlockSpec((_B, _D), lambda b, e: (b, 0)),
    )
    return pl.pallas_call(
        _grouped_body,
        grid_spec=grid_spec,
        out_shape=jax.ShapeDtypeStruct((_P, _D), jnp.float32),
        compiler_params=pltpu.CompilerParams(
            dimension_semantics=("arbitrary",),
        ),
    )(e_of_b, xs, W_gate, W1, W2t)


def _sc_combine_body(
    ys_hbm, d0_hbm, d1_hbm, g0_hbm, g1_hbm, out_hbm,
    i0_v, i1_v, ga_v, gb_v, r0_v, r1_v, sem,
):
    wid = lax.axis_index("s") * 2 + lax.axis_index("c")
    nchunk = _TPW // 32
    for c in range(nchunk):
        base = wid * _TPW + c * 32
        pltpu.sync_copy(d0_hbm.at[pl.ds(base, 32)], i0_v)
        pltpu.sync_copy(d1_hbm.at[pl.ds(base, 32)], i1_v)
        pltpu.sync_copy(g0_hbm.at[pl.ds(base, 32)], ga_v)
        pltpu.sync_copy(g1_hbm.at[pl.ds(base, 32)], gb_v)
        pltpu.async_copy(ys_hbm.at[i0_v], r0_v, sem).wait()
        pltpu.async_copy(ys_hbm.at[i1_v], r1_v, sem).wait()
        ga_lo, ga_hi = ga_v[pl.ds(0, 16)], ga_v[pl.ds(16, 16)]
        gb_lo, gb_hi = gb_v[pl.ds(0, 16)], gb_v[pl.ds(16, 16)]
        for t in range(32):
            a = (ga_lo if t < 16 else ga_hi)[t % 16]
            b = (gb_lo if t < 16 else gb_hi)[t % 16]

            def body(j, _, t=t, a=a, b=b):
                v = r0_v[t, pl.ds(j * 16, 16)] * a + r1_v[t, pl.ds(j * 16, 16)] * b
                r0_v[t, pl.ds(j * 16, 16)] = v
                return 0

            lax.fori_loop(0, _D // 16, body, 0)
        pltpu.sync_copy(r0_v, out_hbm.at[pl.ds(base, 32)])


_sc_combine = functools.partial(
    pl.kernel,
    mesh=plsc.VectorSubcoreMesh(core_axis_name="c", subcore_axis_name="s"),
    out_type=jax.ShapeDtypeStruct((_T, _D), jnp.float32),
    scratch_types=[
        pltpu.VMEM((32,), jnp.int32),
        pltpu.VMEM((32,), jnp.int32),
        pltpu.VMEM((32,), jnp.float32),
        pltpu.VMEM((32,), jnp.float32),
        pltpu.VMEM((32, _D), jnp.float32),
        pltpu.VMEM((32, _D), jnp.float32),
        pltpu.SemaphoreType.DMA,
    ],
)(_sc_combine_body)


def kernel(x, Wg, W_gate, W1, W2):
    tw, mask = _router(x, Wg)

    mi = mask.astype(jnp.int32)
    cnt = jnp.sum(mi, axis=0)
    rank = jnp.cumsum(mi, axis=0) - mi
    nblk = (cnt + _B - 1) // _B
    blk_off = jnp.concatenate(
        [jnp.zeros((1,), jnp.int32), jnp.cumsum(nblk).astype(jnp.int32)]
    )
    pad_off = blk_off * _B
    dst = pad_off[:_E][None, :] + rank
    dst_safe = jnp.where(mi > 0, dst, _P)

    tok_ids = jnp.broadcast_to(
        jnp.arange(_T, dtype=jnp.int32)[:, None], (_T, _E)
    )
    src_tok = (
        jnp.zeros((_P,), jnp.int32)
        .at[dst_safe.reshape(-1)]
        .set(tok_ids.reshape(-1), mode="drop")
    )

    bids = jnp.arange(_G, dtype=jnp.int32)
    e_of_b = jnp.sum(
        (bids[:, None] >= blk_off[None, 1:_E]).astype(jnp.int32), axis=1
    )

    d0 = jnp.min(dst_safe, axis=1).astype(jnp.int32)
    d1 = (jnp.sum(jnp.where(mi > 0, dst, 0), axis=1) - d0).astype(jnp.int32)
    g0 = jnp.sum(jnp.where(dst_safe == d0[:, None], tw, 0.0), axis=1)
    g1 = jnp.sum(jnp.where(dst_safe == d1[:, None], tw, 0.0), axis=1)

    xs = _sc_gather(src_tok, x)
    ys = _grouped(xs, W_gate, W1, W2.transpose(0, 2, 1), e_of_b)
    return _sc_combine(ys, d0, d1, g0, g1)


# speedup vs baseline: 1.0747x; 1.0747x over previous
import functools

import jax
import jax.numpy as jnp
from jax import lax
from jax.experimental import pallas as pl
from jax.experimental.pallas import tpu as pltpu
from jax.experimental.pallas import tpu_sc as plsc

_NEG = -1e30

_T = 2048
_D = 1024
_E = 4
_H = 1368
_B = 256
_G = _T * 2 // _B + (_E - 1)
_P = _G * _B

_NW = 32
_RPW = _P // _NW
_TPW = _T // _NW


def _silu(v):
    return v * (1.0 / (1.0 + jnp.exp(-v)))


def _router_body(x_ref, wg_ref, tw_ref, mask_ref):
    x = x_ref[...]
    logits = jnp.dot(x, wg_ref[...].T, preferred_element_type=jnp.float32)
    m = jnp.max(logits, axis=-1, keepdims=True)
    p = jnp.exp(logits - m)
    p = p / jnp.sum(p, axis=-1, keepdims=True)
    lane = jax.lax.broadcasted_iota(jnp.int32, p.shape, 1)
    p1 = jnp.max(p, axis=-1, keepdims=True)
    i1 = jnp.argmax(p, axis=-1)
    oh1 = (lane == i1[:, None]).astype(jnp.float32)
    pm = jnp.where(oh1 > 0, _NEG, p)
    p2 = jnp.max(pm, axis=-1, keepdims=True)
    i2 = jnp.argmax(pm, axis=-1)
    oh2 = (lane == i2[:, None]).astype(jnp.float32)
    denom = p1 + p2 + 1e-8
    tw_ref[...] = (oh1 * p1 + oh2 * p2) / denom
    mask_ref[...] = oh1 + oh2


def _router(x, Wg):
    return pl.pallas_call(
        _router_body,
        out_shape=(
            jax.ShapeDtypeStruct((_T, _E), jnp.float32),
            jax.ShapeDtypeStruct((_T, _E), jnp.float32),
        ),
    )(x, Wg)


_GCHUNKS = ((0, 40), (40, 40), (80, 40), (120, 32))


def _sc_gather_body(tok_hbm, x_hbm, xs_hbm, idx_v, b0_v, b1_v, gsem, s0, s1):
    wid = lax.axis_index("s") * 2 + lax.axis_index("c")
    base = wid * _RPW
    pltpu.sync_copy(tok_hbm.at[pl.ds(base, _RPW)], idx_v)
    bufs = (b0_v, b1_v)
    ssems = (s0, s1)
    pending = [None, None]
    for c, (off, n) in enumerate(_GCHUNKS):
        slot = c % 2
        if pending[slot] is not None:
            pending[slot].wait()
        pltpu.async_copy(
            x_hbm.at[idx_v.at[pl.ds(off, n)]],
            bufs[slot].at[pl.ds(0, n)],
            gsem,
        ).wait()
        pending[slot] = pltpu.async_copy(
            bufs[slot].at[pl.ds(0, n)],
            xs_hbm.at[pl.ds(base + off, n)],
            ssems[slot],
        )
    for p in pending:
        p.wait()


_sc_gather = functools.partial(
    pl.kernel,
    mesh=plsc.VectorSubcoreMesh(core_axis_name="c", subcore_axis_name="s"),
    out_type=jax.ShapeDtypeStruct((_P, _D), jnp.float32),
    scratch_types=[
        pltpu.VMEM((_RPW,), jnp.int32),
        pltpu.VMEM((40, _D), jnp.float32),
        pltpu.VMEM((40, _D), jnp.float32),
        pltpu.SemaphoreType.DMA,
        pltpu.SemaphoreType.DMA,
        pltpu.SemaphoreType.DMA,
    ],
)(_sc_gather_body)


def _grouped_body(e_ref, xs_ref, gw_ref, wgate_ref, w1_ref, w2t_ref, ys_ref):
    del e_ref
    xb = xs_ref[...]
    g = _silu(jnp.dot(xb, wgate_ref[0].T, preferred_element_type=jnp.float32))
    u = jnp.dot(xb, w1_ref[0].T, preferred_element_type=jnp.float32)
    y = jnp.dot(g * u, w2t_ref[0], preferred_element_type=jnp.float32)
    ys_ref[...] = y * gw_ref[...]


def _grouped(xs, gw, W_gate, W1, W2t, e_of_b):
    grid_spec = pltpu.PrefetchScalarGridSpec(
        num_scalar_prefetch=1,
        grid=(_G,),
        in_specs=[
            pl.BlockSpec((_B, _D), lambda b, e: (b, 0)),
            pl.BlockSpec((_B, 1), lambda b, e: (b, 0)),
            pl.BlockSpec((1, _H, _D), lambda b, e: (e[b], 0, 0)),
            pl.BlockSpec((1, _H, _D), lambda b, e: (e[b], 0, 0)),
            pl.BlockSpec((1, _H, _D), lambda b, e: (e[b], 0, 0)),
        ],
        out_specs=pl.BlockSpec((_B, _D), lambda b, e: (b, 0)),
    )
    return pl.pallas_call(
        _grouped_body,
        grid_spec=grid_spec,
        out_shape=jax.ShapeDtypeStruct((_P, _D), jnp.float32),
        compiler_params=pltpu.CompilerParams(
            dimension_semantics=("arbitrary",),
        ),
    )(e_of_b, xs, gw.reshape(_P, 1), W_gate, W1, W2t)


def _sc_combine_body(
    ys_hbm, d0_hbm, d1_hbm, y0_hbm, y1_hbm,
    i0_v, i1_v, b0_v, b1_v, gsem, s0, s1,
):
    wid = lax.axis_index("s") * 2 + lax.axis_index("c")
    base = wid * _TPW
    pltpu.sync_copy(d0_hbm.at[pl.ds(base, _TPW)], i0_v)
    pltpu.sync_copy(d1_hbm.at[pl.ds(base, _TPW)], i1_v)
    bufs = (b0_v, b1_v)
    ssems = (s0, s1)
    pending = [None, None]
    work = (
        (i0_v, 0, y0_hbm), (i0_v, 32, y0_hbm),
        (i1_v, 0, y1_hbm), (i1_v, 32, y1_hbm),
    )
    for c, (idx_v, off, dst_hbm) in enumerate(work):
        slot = c % 2
        if pending[slot] is not None:
            pending[slot].wait()
        pltpu.async_copy(
            ys_hbm.at[idx_v.at[pl.ds(off, 32)]], bufs[slot], gsem
        ).wait()
        pending[slot] = pltpu.async_copy(
            bufs[slot], dst_hbm.at[pl.ds(base + off, 32)], ssems[slot]
        )
    for p in pending:
        p.wait()


_sc_combine = functools.partial(
    pl.kernel,
    mesh=plsc.VectorSubcoreMesh(core_axis_name="c", subcore_axis_name="s"),
    out_type=(
        jax.ShapeDtypeStruct((_T, _D), jnp.float32),
        jax.ShapeDtypeStruct((_T, _D), jnp.float32),
    ),
    scratch_types=[
        pltpu.VMEM((_TPW,), jnp.int32),
        pltpu.VMEM((_TPW,), jnp.int32),
        pltpu.VMEM((32, _D), jnp.float32),
        pltpu.VMEM((32, _D), jnp.float32),
        pltpu.SemaphoreType.DMA,
        pltpu.SemaphoreType.DMA,
        pltpu.SemaphoreType.DMA,
    ],
)(_sc_combine_body)


def _add_body(a_ref, b_ref, o_ref):
    o_ref[...] = a_ref[...] + b_ref[...]


def _final_add(y0, y1):
    return pl.pallas_call(
        _add_body,
        grid=(8,),
        in_specs=[
            pl.BlockSpec((_T // 8, _D), lambda i: (i, 0)),
            pl.BlockSpec((_T // 8, _D), lambda i: (i, 0)),
        ],
        out_specs=pl.BlockSpec((_T // 8, _D), lambda i: (i, 0)),
        out_shape=jax.ShapeDtypeStruct((_T, _D), jnp.float32),
    )(y0, y1)


def kernel(x, Wg, W_gate, W1, W2):
    tw, mask = _router(x, Wg)

    mi = mask.astype(jnp.int32)
    cnt = jnp.sum(mi, axis=0)
    rank = jnp.cumsum(mi, axis=0) - mi
    nblk = (cnt + _B - 1) // _B
    blk_off = jnp.concatenate(
        [jnp.zeros((1,), jnp.int32), jnp.cumsum(nblk).astype(jnp.int32)]
    )
    pad_off = blk_off * _B
    dst = pad_off[:_E][None, :] + rank
    dst_safe = jnp.where(mi > 0, dst, _P)

    tok_ids = jnp.broadcast_to(
        jnp.arange(_T, dtype=jnp.int32)[:, None], (_T, _E)
    )
    src_tok = (
        jnp.zeros((_P,), jnp.int32)
        .at[dst_safe.reshape(-1)]
        .set(tok_ids.reshape(-1), mode="drop")
    )
    gw_p = (
        jnp.zeros((_P,), jnp.float32)
        .at[dst_safe.reshape(-1)]
        .set(tw.reshape(-1), mode="drop")
    )

    bids = jnp.arange(_G, dtype=jnp.int32)
    e_of_b = jnp.sum(
        (bids[:, None] >= blk_off[None, 1:_E]).astype(jnp.int32), axis=1
    )

    d0 = jnp.min(dst_safe, axis=1).astype(jnp.int32)
    d1 = (jnp.sum(jnp.where(mi > 0, dst, 0), axis=1) - d0).astype(jnp.int32)

    xs = _sc_gather(src_tok, x)
    ys = _grouped(xs, gw_p, W_gate, W1, W2.transpose(0, 2, 1), e_of_b)
    y0, y1 = _sc_combine(ys, d0, d1)
    return _final_add(y0, y1)
